# baseline (speedup 1.0000x reference)
import functools
import jax
import jax.numpy as jnp
from jax import lax
from jax.experimental import pallas as pl
from jax.experimental.pallas import tpu as pltpu
from jax.experimental.pallas import tpu_sc as plsc

_NC = 2
_NS = 16
_NW = _NC * _NS


def _make_deg(n_pad, e, d, k):
    epw = e // _NW
    nchunks = epw // k
    rpt = n_pad // _NS
    mesh = plsc.VectorSubcoreMesh(core_axis_name="c", subcore_axis_name="s")

    @functools.partial(
        pl.kernel,
        out_type=jax.ShapeDtypeStruct((_NC, n_pad, d), jnp.float32),
        mesh=mesh,
        scratch_types=[
            pltpu.VMEM((k,), jnp.int32),
            pltpu.VMEM((k, d), jnp.float32),
            pltpu.VMEM_SHARED((n_pad, d), jnp.float32),
        ],
    )
    def deg_kernel(dst_hbm, out_hbm, didx, ones, acc):
        c = lax.axis_index("c")
        s = lax.axis_index("s")
        wid = c * _NS + s

        def zrow(i, _):
            def zcol(j, _):
                ones[i, pl.ds(j * 16, 16)] = jnp.zeros((16,), jnp.float32)
                return 0
            return lax.fori_loop(0, d // 16, zcol, 0)
        lax.fori_loop(0, k, zrow, 0)

        nfull = rpt // k
        rem = rpt - nfull * k
        r0 = s * rpt

        def zcopy(i, _):
            pltpu.sync_copy(ones, acc.at[pl.ds(r0 + i * k, k)])
            return 0
        lax.fori_loop(0, nfull, zcopy, 0)
        if rem:
            pltpu.sync_copy(ones.at[pl.ds(0, rem)],
                            acc.at[pl.ds(r0 + nfull * k, rem)])

        one_row = jnp.where(lax.iota(jnp.int32, 16) == 0, 1.0, 0.0)

        def fill(i, _):
            ones[i, pl.ds(0, 16)] = one_row
            return 0
        lax.fori_loop(0, k, fill, 0)
        plsc.subcore_barrier()

        base = wid * epw

        def chunk(i, _):
            off = pl.multiple_of(base + i * k, 8)
            pltpu.sync_copy(dst_hbm.at[pl.ds(off, k)], didx)
            pltpu.sync_copy(ones, acc.at[didx], add=True)
            return 0
        lax.fori_loop(0, nchunks, chunk, 0)
        plsc.subcore_barrier()

        def ocopy(i, _):
            pltpu.sync_copy(acc.at[pl.ds(r0 + i * k, k)], ones)
            pltpu.sync_copy(ones, out_hbm.at[c, pl.ds(r0 + i * k, k)])
            return 0
        lax.fori_loop(0, nfull, ocopy, 0)
        if rem:
            pltpu.sync_copy(acc.at[pl.ds(r0 + nfull * k, rem)],
                            ones.at[pl.ds(0, rem)])
            pltpu.sync_copy(ones.at[pl.ds(0, rem)],
                            out_hbm.at[c, pl.ds(r0 + nfull * k, rem)])

    return deg_kernel


def _make_agg(n_acc, e, d, k):
    n = n_acc
    epw = e // _NW
    nchunks = epw // k
    rpt = n // _NS
    mesh = plsc.VectorSubcoreMesh(core_axis_name="c", subcore_axis_name="s")

    cpt = nchunks
    assert cpt % 2 == 1

    @functools.partial(
        pl.kernel,
        out_type=jax.ShapeDtypeStruct((_NC, n, d), jnp.float32),
        mesh=mesh,
        scratch_types=[
            pltpu.VMEM((k,), jnp.int32),
            pltpu.VMEM((k,), jnp.int32),
            pltpu.VMEM((k,), jnp.int32),
            pltpu.VMEM((k,), jnp.int32),
            pltpu.VMEM((k, d), jnp.float32),
            pltpu.VMEM((k, d), jnp.float32),
            pltpu.VMEM_SHARED((n, d), jnp.float32),
            pltpu.SemaphoreType.DMA,
            pltpu.SemaphoreType.DMA,
        ],
    )
    def agg_kernel(y_hbm, src_hbm, dst_hbm, out_hbm, sb0, db0, sb1, db1,
                   buf0, buf1, acc, semg0, semg1):
        c = lax.axis_index("c")
        s = lax.axis_index("s")
        wid = c * _NS + s

        def zrow(i, _):
            def zcol(j, _):
                buf0[i, pl.ds(j * 16, 16)] = jnp.zeros((16,), jnp.float32)
                return 0
            return lax.fori_loop(0, d // 16, zcol, 0)
        lax.fori_loop(0, k, zrow, 0)

        nfull = rpt // k
        rem = rpt - nfull * k
        r0 = s * rpt

        def zcopy(i, _):
            pltpu.sync_copy(buf0, acc.at[pl.ds(r0 + i * k, k)])
            return 0
        lax.fori_loop(0, nfull, zcopy, 0)
        if rem:
            pltpu.sync_copy(buf0.at[pl.ds(0, rem)],
                            acc.at[pl.ds(r0 + nfull * k, rem)])
        plsc.subcore_barrier()

        base = wid * epw

        def pair(g, _):
            c0 = 2 * g
            o0 = pl.multiple_of(base + c0 * k, 8)
            o1 = pl.multiple_of(base + (c0 + 1) * k, 8)
            pltpu.sync_copy(src_hbm.at[pl.ds(o0, k)], sb0)
            pltpu.sync_copy(dst_hbm.at[pl.ds(o0, k)], db0)
            pltpu.async_copy(y_hbm.at[sb0], buf0, semg0)
            pltpu.sync_copy(src_hbm.at[pl.ds(o1, k)], sb1)
            pltpu.sync_copy(dst_hbm.at[pl.ds(o1, k)], db1)
            pltpu.async_copy(y_hbm.at[sb1], buf1, semg1)
            pltpu.make_async_copy(y_hbm.at[sb0], buf0, semg0).wait()
            pltpu.make_async_copy(y_hbm.at[sb1], buf1, semg1).wait()
            pltpu.sync_copy(buf0, acc.at[db0], add=True)
            pltpu.sync_copy(buf1, acc.at[db1], add=True)
            return 0
        lax.fori_loop(0, (cpt - 1) // 2, pair, 0)
        oe = pl.multiple_of(base + (cpt - 1) * k, 8)
        pltpu.sync_copy(src_hbm.at[pl.ds(oe, k)], sb0)
        pltpu.sync_copy(dst_hbm.at[pl.ds(oe, k)], db0)
        pltpu.async_copy(y_hbm.at[sb0], buf0, semg0)
        pltpu.make_async_copy(y_hbm.at[sb0], buf0, semg0).wait()
        pltpu.sync_copy(buf0, acc.at[db0], add=True)
        plsc.subcore_barrier()

        def ocopy(i, _):
            pltpu.sync_copy(acc.at[pl.ds(r0 + i * k, k)], buf0)
            pltpu.sync_copy(buf0, out_hbm.at[c, pl.ds(r0 + i * k, k)])
            return 0
        lax.fori_loop(0, nfull, ocopy, 0)
        if rem:
            pltpu.sync_copy(acc.at[pl.ds(r0 + nfull * k, rem)],
                            buf0.at[pl.ds(0, rem)])
            pltpu.sync_copy(buf0.at[pl.ds(0, rem)],
                            out_hbm.at[c, pl.ds(r0 + nfull * k, rem)])

    return agg_kernel


def _add_body(a_ref, b_ref, out_ref):
    out_ref[...] = a_ref[...] + b_ref[...]


def _l1_body(x_ref, sx0_ref, sx1_ref, sd0_ref, sd1_ref, d_ref, wt_ref,
             b_ref, out_ref):
    deg = d_ref[...]
    pos = deg > 0.0
    dd = jnp.maximum(deg, 1.0)
    sdc = (sd0_ref[...] + sd1_ref[...])[:, 0:1]
    hd = jnp.where(pos, sdc / dd, deg)
    hx = jnp.where(pos, (sx0_ref[...] + sx1_ref[...]) / dd, x_ref[...])
    hup = jnp.concatenate([hd, hx], axis=1)
    out_ref[...] = jnp.maximum(
        jnp.dot(hup, wt_ref[...], preferred_element_type=jnp.float32)
        + b_ref[...], 0.0)


def _mid_body(h_ref, a0_ref, a1_ref, d_ref, b_ref, wt_ref, out_ref):
    deg = d_ref[...]
    mean = (a0_ref[...] + a1_ref[...]) / jnp.maximum(deg, 1.0)
    hup = jnp.where(deg > 0.0, mean, h_ref[...])
    out_ref[...] = jnp.maximum(
        jnp.dot(hup, wt_ref[...], preferred_element_type=jnp.float32)
        + b_ref[...], 0.0)


def _final_body(n, ngrid, h_ref, a0_ref, a1_ref, d_ref, b_ref, wt_ref,
                wc1t_ref, bc1_ref, wc2t_ref, bc2_ref, out_ref, acc_ref):
    i = pl.program_id(0)
    deg = d_ref[...]
    mean = (a0_ref[...] + a1_ref[...]) / jnp.maximum(deg, 1.0)
    hup = jnp.where(deg > 0.0, mean, h_ref[...])
    h3 = jnp.maximum(
        jnp.dot(hup, wt_ref[...], preferred_element_type=jnp.float32)
        + b_ref[...], 0.0)
    part = jnp.sum(h3, axis=0, keepdims=True)

    @pl.when(i == 0)
    def _():
        acc_ref[...] = part

    @pl.when(i > 0)
    def _():
        acc_ref[...] += part

    @pl.when(i == ngrid - 1)
    def _():
        hg = acc_ref[...] / float(n)
        hg = jnp.dot(hg, wc1t_ref[...],
                     preferred_element_type=jnp.float32) + bc1_ref[...]
        hg = jnp.dot(hg, wc1t_ref[...],
                     preferred_element_type=jnp.float32) + bc1_ref[...]
        out_ref[...] = jnp.dot(hg, wc2t_ref[...],
                               preferred_element_type=jnp.float32) + bc2_ref[...]


def _row_spec(blk, d):
    return pl.BlockSpec((blk, d), lambda i: (i, 0))


def _full_spec(shape):
    return pl.BlockSpec(shape, lambda i: tuple(0 for _ in shape))


def kernel(x, edge_index, W1, b1, W2, b2, W3, b3, Wc1, bc1, Wc2, bc2):
    n, d = x.shape
    e = edge_index.shape[1]
    h = W1.shape[0]
    src = edge_index[0]
    dst = edge_index[1]

    n_pad = ((n + (8 * _NS) - 1) // (8 * _NS)) * (8 * _NS)
    k = 80

    deg_p = _make_deg(n_pad, e, h, k)(dst)

    blk = 1000
    ngrid = n // blk
    row = functools.partial(_row_spec, blk)
    dspec = pl.BlockSpec((blk, 1), lambda i: (i, 0))

    degmat = pl.pallas_call(
        _add_body,
        grid=(ngrid,),
        in_specs=[row(h), row(h)],
        out_specs=row(h),
        out_shape=jax.ShapeDtypeStruct((n, h), jnp.float32),
    )(deg_p[0], deg_p[1])
    dcol = degmat[:, 0:1]

    agg = _make_agg(n_pad, e, h, k)
    sx = agg(x, src, dst)
    sd = agg(degmat, src, dst)

    h1 = pl.pallas_call(
        _l1_body,
        grid=(ngrid,),
        in_specs=[row(d), row(h), row(h), row(h), row(h), dspec,
                  _full_spec((d + 1, h)), _full_spec((1, h))],
        out_specs=row(h),
        out_shape=jax.ShapeDtypeStruct((n, h), jnp.float32),
    )(x, sx[0], sx[1], sd[0], sd[1], dcol, W1.T, b1.reshape(1, h))

    mid = pl.pallas_call(
        _mid_body,
        grid=(ngrid,),
        in_specs=[row(h), row(h), row(h), dspec,
                  _full_spec((1, h)), _full_spec((h, h))],
        out_specs=row(h),
        out_shape=jax.ShapeDtypeStruct((n, h), jnp.float32),
    )

    a = agg(h1, src, dst)
    h2 = mid(h1, a[0], a[1], dcol, b2.reshape(1, h), W2.T)
    a = agg(h2, src, dst)

    out = pl.pallas_call(
        functools.partial(_final_body, n, ngrid),
        grid=(ngrid,),
        in_specs=[row(h), row(h), row(h), dspec, _full_spec((1, h)),
                  _full_spec((h, h)),
                  _full_spec((h, h)), _full_spec((1, h)),
                  _full_spec((h, 1)), _full_spec((1, 1))],
        out_specs=_full_spec((1, 1)),
        out_shape=jax.ShapeDtypeStruct((1, 1), jnp.float32),
        scratch_shapes=[pltpu.VMEM((1, h), jnp.float32)],
    )(h2, a[0], a[1], dcol, b3.reshape(1, h), W3.T,
      Wc1.T, bc1.reshape(1, h), Wc2.T, bc2.reshape(1, 1))

    return out


# speedup vs baseline: 1.2701x; 1.2701x over previous
import functools
import jax
import jax.numpy as jnp
from jax import lax
from jax.experimental import pallas as pl
from jax.experimental.pallas import tpu as pltpu
from jax.experimental.pallas import tpu_sc as plsc

_NC = 2
_NS = 16
_NW = _NC * _NS


def _make_deg(n_pad, e, d, k):
    epw = e // _NW
    nchunks = epw // k
    rpt = n_pad // _NS
    mesh = plsc.VectorSubcoreMesh(core_axis_name="c", subcore_axis_name="s")

    @functools.partial(
        pl.kernel,
        out_type=jax.ShapeDtypeStruct((_NC, n_pad, d), jnp.float32),
        mesh=mesh,
        scratch_types=[
            pltpu.VMEM((k,), jnp.int32),
            pltpu.VMEM((k, d), jnp.float32),
            pltpu.VMEM_SHARED((n_pad, d), jnp.float32),
        ],
    )
    def deg_kernel(dst_hbm, out_hbm, didx, ones, acc):
        c = lax.axis_index("c")
        s = lax.axis_index("s")
        wid = c * _NS + s

        def zrow(i, _):
            def zcol(j, _):
                ones[i, pl.ds(j * 16, 16)] = jnp.zeros((16,), jnp.float32)
                return 0
            return lax.fori_loop(0, d // 16, zcol, 0)
        lax.fori_loop(0, k, zrow, 0)

        nfull = rpt // k
        rem = rpt - nfull * k
        r0 = s * rpt

        def zcopy(i, _):
            pltpu.sync_copy(ones, acc.at[pl.ds(r0 + i * k, k)])
            return 0
        lax.fori_loop(0, nfull, zcopy, 0)
        if rem:
            pltpu.sync_copy(ones.at[pl.ds(0, rem)],
                            acc.at[pl.ds(r0 + nfull * k, rem)])

        one_row = jnp.where(lax.iota(jnp.int32, 16) == 0, 1.0, 0.0)

        def fill(i, _):
            ones[i, pl.ds(0, 16)] = one_row
            return 0
        lax.fori_loop(0, k, fill, 0)
        plsc.subcore_barrier()

        base = wid * epw

        def chunk(i, _):
            off = pl.multiple_of(base + i * k, 8)
            pltpu.sync_copy(dst_hbm.at[pl.ds(off, k)], didx)
            pltpu.sync_copy(ones, acc.at[didx], add=True)
            return 0
        lax.fori_loop(0, nchunks, chunk, 0)
        plsc.subcore_barrier()

        def ocopy(i, _):
            pltpu.sync_copy(acc.at[pl.ds(r0 + i * k, k)], ones)
            pltpu.sync_copy(ones, out_hbm.at[c, pl.ds(r0 + i * k, k)])
            return 0
        lax.fori_loop(0, nfull, ocopy, 0)
        if rem:
            pltpu.sync_copy(acc.at[pl.ds(r0 + nfull * k, rem)],
                            ones.at[pl.ds(0, rem)])
            pltpu.sync_copy(ones.at[pl.ds(0, rem)],
                            out_hbm.at[c, pl.ds(r0 + nfull * k, rem)])

    return deg_kernel


def _make_agg(n_acc, e, d, k):
    n = n_acc
    epw = e // _NW
    nchunks = epw // k
    rpt = n // _NS
    mesh = plsc.VectorSubcoreMesh(core_axis_name="c", subcore_axis_name="s")

    cpt = nchunks
    assert cpt % 2 == 1

    @functools.partial(
        pl.kernel,
        out_type=jax.ShapeDtypeStruct((_NC, n, d), jnp.float32),
        mesh=mesh,
        scratch_types=[
            pltpu.VMEM((k,), jnp.int32),
            pltpu.VMEM((k,), jnp.int32),
            pltpu.VMEM((k,), jnp.int32),
            pltpu.VMEM((k,), jnp.int32),
            pltpu.VMEM((k, d), jnp.float32),
            pltpu.VMEM((k, d), jnp.float32),
            pltpu.VMEM_SHARED((n, d), jnp.float32),
            pltpu.SemaphoreType.DMA,
            pltpu.SemaphoreType.DMA,
        ],
    )
    def agg_kernel(y_hbm, src_hbm, dst_hbm, out_hbm, sb0, db0, sb1, db1,
                   buf0, buf1, acc, semg0, semg1):
        c = lax.axis_index("c")
        s = lax.axis_index("s")
        wid = c * _NS + s

        def zrow(i, _):
            def zcol(j, _):
                buf0[i, pl.ds(j * 16, 16)] = jnp.zeros((16,), jnp.float32)
                return 0
            return lax.fori_loop(0, d // 16, zcol, 0)
        lax.fori_loop(0, k, zrow, 0)

        nfull = rpt // k
        rem = rpt - nfull * k
        r0 = s * rpt

        def zcopy(i, _):
            pltpu.sync_copy(buf0, acc.at[pl.ds(r0 + i * k, k)])
            return 0
        lax.fori_loop(0, nfull, zcopy, 0)
        if rem:
            pltpu.sync_copy(buf0.at[pl.ds(0, rem)],
                            acc.at[pl.ds(r0 + nfull * k, rem)])
        plsc.subcore_barrier()

        base = wid * epw

        def idx_load(cc, sb, db):
            off = pl.multiple_of(base + cc * k, 8)
            pltpu.sync_copy(src_hbm.at[pl.ds(off, k)], sb)
            pltpu.sync_copy(dst_hbm.at[pl.ds(off, k)], db)

        idx_load(0, sb0, db0)
        pltpu.async_copy(y_hbm.at[sb0], buf0, semg0)

        def pair(g, _):
            c0 = 2 * g
            idx_load(c0 + 1, sb1, db1)
            pltpu.async_copy(y_hbm.at[sb1], buf1, semg1)
            pltpu.make_async_copy(y_hbm.at[sb0], buf0, semg0).wait()
            pltpu.sync_copy(buf0, acc.at[db0], add=True)
            idx_load(c0 + 2, sb0, db0)
            pltpu.async_copy(y_hbm.at[sb0], buf0, semg0)
            pltpu.make_async_copy(y_hbm.at[sb1], buf1, semg1).wait()
            pltpu.sync_copy(buf1, acc.at[db1], add=True)
            return 0
        lax.fori_loop(0, (cpt - 1) // 2, pair, 0)
        pltpu.make_async_copy(y_hbm.at[sb0], buf0, semg0).wait()
        pltpu.sync_copy(buf0, acc.at[db0], add=True)
        plsc.subcore_barrier()

        def ocopy(i, _):
            pltpu.sync_copy(acc.at[pl.ds(r0 + i * k, k)], buf0)
            pltpu.sync_copy(buf0, out_hbm.at[c, pl.ds(r0 + i * k, k)])
            return 0
        lax.fori_loop(0, nfull, ocopy, 0)
        if rem:
            pltpu.sync_copy(acc.at[pl.ds(r0 + nfull * k, rem)],
                            buf0.at[pl.ds(0, rem)])
            pltpu.sync_copy(buf0.at[pl.ds(0, rem)],
                            out_hbm.at[c, pl.ds(r0 + nfull * k, rem)])

    return agg_kernel


def _add_body(a_ref, b_ref, out_ref):
    out_ref[...] = a_ref[...] + b_ref[...]


def _l1_body(x_ref, sx0_ref, sx1_ref, sd0_ref, sd1_ref, d_ref, wt_ref,
             b_ref, out_ref):
    deg = d_ref[...]
    pos = deg > 0.0
    dd = jnp.maximum(deg, 1.0)
    sdc = (sd0_ref[...] + sd1_ref[...])[:, 0:1]
    hd = jnp.where(pos, sdc / dd, deg)
    hx = jnp.where(pos, (sx0_ref[...] + sx1_ref[...]) / dd, x_ref[...])
    hup = jnp.concatenate([hd, hx], axis=1)
    out_ref[...] = jnp.maximum(
        jnp.dot(hup, wt_ref[...], preferred_element_type=jnp.float32)
        + b_ref[...], 0.0)


def _mid_body(h_ref, a0_ref, a1_ref, d_ref, b_ref, wt_ref, out_ref):
    deg = d_ref[...]
    mean = (a0_ref[...] + a1_ref[...]) / jnp.maximum(deg, 1.0)
    hup = jnp.where(deg > 0.0, mean, h_ref[...])
    out_ref[...] = jnp.maximum(
        jnp.dot(hup, wt_ref[...], preferred_element_type=jnp.float32)
        + b_ref[...], 0.0)


def _final_body(n, ngrid, h_ref, a0_ref, a1_ref, d_ref, b_ref, wt_ref,
                wc1t_ref, bc1_ref, wc2t_ref, bc2_ref, out_ref, acc_ref):
    i = pl.program_id(0)
    deg = d_ref[...]
    mean = (a0_ref[...] + a1_ref[...]) / jnp.maximum(deg, 1.0)
    hup = jnp.where(deg > 0.0, mean, h_ref[...])
    h3 = jnp.maximum(
        jnp.dot(hup, wt_ref[...], preferred_element_type=jnp.float32)
        + b_ref[...], 0.0)
    part = jnp.sum(h3, axis=0, keepdims=True)

    @pl.when(i == 0)
    def _():
        acc_ref[...] = part

    @pl.when(i > 0)
    def _():
        acc_ref[...] += part

    @pl.when(i == ngrid - 1)
    def _():
        hg = acc_ref[...] / float(n)
        hg = jnp.dot(hg, wc1t_ref[...],
                     preferred_element_type=jnp.float32) + bc1_ref[...]
        hg = jnp.dot(hg, wc1t_ref[...],
                     preferred_element_type=jnp.float32) + bc1_ref[...]
        out_ref[...] = jnp.dot(hg, wc2t_ref[...],
                               preferred_element_type=jnp.float32) + bc2_ref[...]


def _row_spec(blk, d):
    return pl.BlockSpec((blk, d), lambda i: (i, 0))


def _full_spec(shape):
    return pl.BlockSpec(shape, lambda i: tuple(0 for _ in shape))


def kernel(x, edge_index, W1, b1, W2, b2, W3, b3, Wc1, bc1, Wc2, bc2):
    n, d = x.shape
    e = edge_index.shape[1]
    h = W1.shape[0]
    src = edge_index[0]
    dst = edge_index[1]

    n_pad = ((n + (8 * _NS) - 1) // (8 * _NS)) * (8 * _NS)
    k = 80

    deg_p = _make_deg(n_pad, e, h, k)(dst)

    blk = 1000
    ngrid = n // blk
    row = functools.partial(_row_spec, blk)
    dspec = pl.BlockSpec((blk, 1), lambda i: (i, 0))

    degmat = pl.pallas_call(
        _add_body,
        grid=(ngrid,),
        in_specs=[row(h), row(h)],
        out_specs=row(h),
        out_shape=jax.ShapeDtypeStruct((n, h), jnp.float32),
    )(deg_p[0], deg_p[1])
    dcol = degmat[:, 0:1]

    agg = _make_agg(n_pad, e, h, k)
    sx = agg(x, src, dst)
    sd = agg(degmat, src, dst)

    h1 = pl.pallas_call(
        _l1_body,
        grid=(ngrid,),
        in_specs=[row(d), row(h), row(h), row(h), row(h), dspec,
                  _full_spec((d + 1, h)), _full_spec((1, h))],
        out_specs=row(h),
        out_shape=jax.ShapeDtypeStruct((n, h), jnp.float32),
    )(x, sx[0], sx[1], sd[0], sd[1], dcol, W1.T, b1.reshape(1, h))

    mid = pl.pallas_call(
        _mid_body,
        grid=(ngrid,),
        in_specs=[row(h), row(h), row(h), dspec,
                  _full_spec((1, h)), _full_spec((h, h))],
        out_specs=row(h),
        out_shape=jax.ShapeDtypeStruct((n, h), jnp.float32),
    )

    a = agg(h1, src, dst)
    h2 = mid(h1, a[0], a[1], dcol, b2.reshape(1, h), W2.T)
    a = agg(h2, src, dst)

    out = pl.pallas_call(
        functools.partial(_final_body, n, ngrid),
        grid=(ngrid,),
        in_specs=[row(h), row(h), row(h), dspec, _full_spec((1, h)),
                  _full_spec((h, h)),
                  _full_spec((h, h)), _full_spec((1, h)),
                  _full_spec((h, 1)), _full_spec((1, 1))],
        out_specs=_full_spec((1, 1)),
        out_shape=jax.ShapeDtypeStruct((1, 1), jnp.float32),
        scratch_shapes=[pltpu.VMEM((1, h), jnp.float32)],
    )(h2, a[0], a[1], dcol, b3.reshape(1, h), W3.T,
      Wc1.T, bc1.reshape(1, h), Wc2.T, bc2.reshape(1, 1))

    return out


# speedup vs baseline: 1.3413x; 1.0560x over previous
import functools
import jax
import jax.numpy as jnp
from jax import lax
from jax.experimental import pallas as pl
from jax.experimental.pallas import tpu as pltpu
from jax.experimental.pallas import tpu_sc as plsc

_NC = 2
_NS = 16
_NW = _NC * _NS


def _make_deg(n_pad, e, d, k):
    epw = e // _NW
    nchunks = epw // k
    rpt = n_pad // _NS
    mesh = plsc.VectorSubcoreMesh(core_axis_name="c", subcore_axis_name="s")

    @functools.partial(
        pl.kernel,
        out_type=jax.ShapeDtypeStruct((_NC, n_pad, d), jnp.float32),
        mesh=mesh,
        scratch_types=[
            pltpu.VMEM((k,), jnp.int32),
            pltpu.VMEM((k,), jnp.int32),
            pltpu.VMEM((k, d), jnp.float32),
            pltpu.VMEM_SHARED((n_pad, d), jnp.float32),
            pltpu.SemaphoreType.DMA,
            pltpu.SemaphoreType.DMA,
        ],
    )
    def deg_kernel(dst_hbm, out_hbm, db0, db1, ones, acc, semd0, semd1):
        c = lax.axis_index("c")
        s = lax.axis_index("s")
        wid = c * _NS + s

        def zrow(i, _):
            def zcol(j, _):
                ones[i, pl.ds(j * 16, 16)] = jnp.zeros((16,), jnp.float32)
                return 0
            return lax.fori_loop(0, d // 16, zcol, 0)
        lax.fori_loop(0, k, zrow, 0)

        nfull = rpt // k
        rem = rpt - nfull * k
        r0 = s * rpt

        def zcopy(i, _):
            pltpu.sync_copy(ones, acc.at[pl.ds(r0 + i * k, k)])
            return 0
        lax.fori_loop(0, nfull, zcopy, 0)
        if rem:
            pltpu.sync_copy(ones.at[pl.ds(0, rem)],
                            acc.at[pl.ds(r0 + nfull * k, rem)])

        one_row = jnp.where(lax.iota(jnp.int32, 16) == 0, 1.0, 0.0)

        def fill(i, _):
            ones[i, pl.ds(0, 16)] = one_row
            return 0
        lax.fori_loop(0, k, fill, 0)
        plsc.subcore_barrier()

        base = wid * epw

        def idx_load(cc, db):
            off = pl.multiple_of(base + cc * k, 8)
            pltpu.sync_copy(dst_hbm.at[pl.ds(off, k)], db)

        idx_load(0, db0)
        pltpu.async_copy(ones, acc.at[db0], semd0, add=True)

        def chunk_pair(g, _):
            c0 = 2 * g
            idx_load(c0 + 1, db1)
            pltpu.async_copy(ones, acc.at[db1], semd1, add=True)
            pltpu.make_async_copy(ones, acc.at[db0], semd0).wait()
            idx_load(c0 + 2, db0)
            pltpu.async_copy(ones, acc.at[db0], semd0, add=True)
            pltpu.make_async_copy(ones, acc.at[db1], semd1).wait()
            return 0
        lax.fori_loop(0, (nchunks - 1) // 2, chunk_pair, 0)
        pltpu.make_async_copy(ones, acc.at[db0], semd0).wait()
        plsc.subcore_barrier()

        def ocopy(i, _):
            pltpu.sync_copy(acc.at[pl.ds(r0 + i * k, k)], ones)
            pltpu.sync_copy(ones, out_hbm.at[c, pl.ds(r0 + i * k, k)])
            return 0
        lax.fori_loop(0, nfull, ocopy, 0)
        if rem:
            pltpu.sync_copy(acc.at[pl.ds(r0 + nfull * k, rem)],
                            ones.at[pl.ds(0, rem)])
            pltpu.sync_copy(ones.at[pl.ds(0, rem)],
                            out_hbm.at[c, pl.ds(r0 + nfull * k, rem)])

    return deg_kernel


def _make_agg(n_acc, e, d, k):
    n = n_acc
    epw = e // _NW
    nchunks = epw // k
    rpt = n // _NS
    mesh = plsc.VectorSubcoreMesh(core_axis_name="c", subcore_axis_name="s")

    cpt = nchunks
    assert cpt % 2 == 1

    @functools.partial(
        pl.kernel,
        out_type=jax.ShapeDtypeStruct((_NC, n, d), jnp.float32),
        mesh=mesh,
        scratch_types=[
            pltpu.VMEM((k,), jnp.int32),
            pltpu.VMEM((k,), jnp.int32),
            pltpu.VMEM((k,), jnp.int32),
            pltpu.VMEM((k,), jnp.int32),
            pltpu.VMEM((k, d), jnp.float32),
            pltpu.VMEM((k, d), jnp.float32),
            pltpu.VMEM_SHARED((n, d), jnp.float32),
            pltpu.SemaphoreType.DMA,
            pltpu.SemaphoreType.DMA,
        ],
    )
    def agg_kernel(y_hbm, src_hbm, dst_hbm, out_hbm, sb0, db0, sb1, db1,
                   buf0, buf1, acc, semg0, semg1):
        c = lax.axis_index("c")
        s = lax.axis_index("s")
        wid = c * _NS + s

        def zrow(i, _):
            def zcol(j, _):
                buf0[i, pl.ds(j * 16, 16)] = jnp.zeros((16,), jnp.float32)
                return 0
            return lax.fori_loop(0, d // 16, zcol, 0)
        lax.fori_loop(0, k, zrow, 0)

        nfull = rpt // k
        rem = rpt - nfull * k
        r0 = s * rpt

        def zcopy(i, _):
            pltpu.sync_copy(buf0, acc.at[pl.ds(r0 + i * k, k)])
            return 0
        lax.fori_loop(0, nfull, zcopy, 0)
        if rem:
            pltpu.sync_copy(buf0.at[pl.ds(0, rem)],
                            acc.at[pl.ds(r0 + nfull * k, rem)])
        plsc.subcore_barrier()

        base = wid * epw

        def idx_load(cc, sb, db):
            off = pl.multiple_of(base + cc * k, 8)
            pltpu.sync_copy(src_hbm.at[pl.ds(off, k)], sb)
            pltpu.sync_copy(dst_hbm.at[pl.ds(off, k)], db)

        idx_load(0, sb0, db0)
        pltpu.async_copy(y_hbm.at[sb0], buf0, semg0)

        def pair(g, _):
            c0 = 2 * g
            idx_load(c0 + 1, sb1, db1)
            pltpu.async_copy(y_hbm.at[sb1], buf1, semg1)
            pltpu.make_async_copy(y_hbm.at[sb0], buf0, semg0).wait()
            pltpu.sync_copy(buf0, acc.at[db0], add=True)
            idx_load(c0 + 2, sb0, db0)
            pltpu.async_copy(y_hbm.at[sb0], buf0, semg0)
            pltpu.make_async_copy(y_hbm.at[sb1], buf1, semg1).wait()
            pltpu.sync_copy(buf1, acc.at[db1], add=True)
            return 0
        lax.fori_loop(0, (cpt - 1) // 2, pair, 0)
        pltpu.make_async_copy(y_hbm.at[sb0], buf0, semg0).wait()
        pltpu.sync_copy(buf0, acc.at[db0], add=True)
        plsc.subcore_barrier()

        def ocopy(i, _):
            pltpu.sync_copy(acc.at[pl.ds(r0 + i * k, k)], buf0)
            pltpu.sync_copy(buf0, out_hbm.at[c, pl.ds(r0 + i * k, k)])
            return 0
        lax.fori_loop(0, nfull, ocopy, 0)
        if rem:
            pltpu.sync_copy(acc.at[pl.ds(r0 + nfull * k, rem)],
                            buf0.at[pl.ds(0, rem)])
            pltpu.sync_copy(buf0.at[pl.ds(0, rem)],
                            out_hbm.at[c, pl.ds(r0 + nfull * k, rem)])

    return agg_kernel


def _add_body(a_ref, b_ref, out_ref):
    out_ref[...] = a_ref[...] + b_ref[...]


def _l1_body(x_ref, sx0_ref, sx1_ref, sd0_ref, sd1_ref, d_ref, wt_ref,
             b_ref, out_ref):
    deg = d_ref[...]
    pos = deg > 0.0
    dd = jnp.maximum(deg, 1.0)
    sdc = (sd0_ref[...] + sd1_ref[...])[:, 0:1]
    hd = jnp.where(pos, sdc / dd, deg)
    hx = jnp.where(pos, (sx0_ref[...] + sx1_ref[...]) / dd, x_ref[...])
    hup = jnp.concatenate([hd, hx], axis=1)
    out_ref[...] = jnp.maximum(
        jnp.dot(hup, wt_ref[...], preferred_element_type=jnp.float32)
        + b_ref[...], 0.0)


def _mid_body(h_ref, a0_ref, a1_ref, d_ref, b_ref, wt_ref, out_ref):
    deg = d_ref[...]
    mean = (a0_ref[...] + a1_ref[...]) / jnp.maximum(deg, 1.0)
    hup = jnp.where(deg > 0.0, mean, h_ref[...])
    out_ref[...] = jnp.maximum(
        jnp.dot(hup, wt_ref[...], preferred_element_type=jnp.float32)
        + b_ref[...], 0.0)


def _final_body(n, ngrid, h_ref, a0_ref, a1_ref, d_ref, b_ref, wt_ref,
                wc1t_ref, bc1_ref, wc2t_ref, bc2_ref, out_ref, acc_ref):
    i = pl.program_id(0)
    deg = d_ref[...]
    mean = (a0_ref[...] + a1_ref[...]) / jnp.maximum(deg, 1.0)
    hup = jnp.where(deg > 0.0, mean, h_ref[...])
    h3 = jnp.maximum(
        jnp.dot(hup, wt_ref[...], preferred_element_type=jnp.float32)
        + b_ref[...], 0.0)
    part = jnp.sum(h3, axis=0, keepdims=True)

    @pl.when(i == 0)
    def _():
        acc_ref[...] = part

    @pl.when(i > 0)
    def _():
        acc_ref[...] += part

    @pl.when(i == ngrid - 1)
    def _():
        hg = acc_ref[...] / float(n)
        hg = jnp.dot(hg, wc1t_ref[...],
                     preferred_element_type=jnp.float32) + bc1_ref[...]
        hg = jnp.dot(hg, wc1t_ref[...],
                     preferred_element_type=jnp.float32) + bc1_ref[...]
        out_ref[...] = jnp.dot(hg, wc2t_ref[...],
                               preferred_element_type=jnp.float32) + bc2_ref[...]


def _row_spec(blk, d):
    return pl.BlockSpec((blk, d), lambda i: (i, 0))


def _full_spec(shape):
    return pl.BlockSpec(shape, lambda i: tuple(0 for _ in shape))


def kernel(x, edge_index, W1, b1, W2, b2, W3, b3, Wc1, bc1, Wc2, bc2):
    n, d = x.shape
    e = edge_index.shape[1]
    h = W1.shape[0]
    src = edge_index[0]
    dst = edge_index[1]

    n_pad = ((n + (8 * _NS) - 1) // (8 * _NS)) * (8 * _NS)
    k = 80

    deg_p = _make_deg(n_pad, e, h, k)(dst)

    blk = 1000
    ngrid = n // blk
    row = functools.partial(_row_spec, blk)
    dspec = pl.BlockSpec((blk, 1), lambda i: (i, 0))

    degmat = pl.pallas_call(
        _add_body,
        grid=(ngrid,),
        in_specs=[row(h), row(h)],
        out_specs=row(h),
        out_shape=jax.ShapeDtypeStruct((n, h), jnp.float32),
    )(deg_p[0], deg_p[1])
    dcol = degmat[:, 0:1]

    agg = _make_agg(n_pad, e, h, k)
    sx = agg(x, src, dst)
    sd = agg(degmat, src, dst)

    h1 = pl.pallas_call(
        _l1_body,
        grid=(ngrid,),
        in_specs=[row(d), row(h), row(h), row(h), row(h), dspec,
                  _full_spec((d + 1, h)), _full_spec((1, h))],
        out_specs=row(h),
        out_shape=jax.ShapeDtypeStruct((n, h), jnp.float32),
    )(x, sx[0], sx[1], sd[0], sd[1], dcol, W1.T, b1.reshape(1, h))

    mid = pl.pallas_call(
        _mid_body,
        grid=(ngrid,),
        in_specs=[row(h), row(h), row(h), dspec,
                  _full_spec((1, h)), _full_spec((h, h))],
        out_specs=row(h),
        out_shape=jax.ShapeDtypeStruct((n, h), jnp.float32),
    )

    a = agg(h1, src, dst)
    h2 = mid(h1, a[0], a[1], dcol, b2.reshape(1, h), W2.T)
    a = agg(h2, src, dst)

    out = pl.pallas_call(
        functools.partial(_final_body, n, ngrid),
        grid=(ngrid,),
        in_specs=[row(h), row(h), row(h), dspec, _full_spec((1, h)),
                  _full_spec((h, h)),
                  _full_spec((h, h)), _full_spec((1, h)),
                  _full_spec((h, 1)), _full_spec((1, 1))],
        out_specs=_full_spec((1, 1)),
        out_shape=jax.ShapeDtypeStruct((1, 1), jnp.float32),
        scratch_shapes=[pltpu.VMEM((1, h), jnp.float32)],
    )(h2, a[0], a[1], dcol, b3.reshape(1, h), W3.T,
      Wc1.T, bc1.reshape(1, h), Wc2.T, bc2.reshape(1, 1))

    return out
